# baseline (speedup 1.0000x reference)
import jax
import jax.numpy as jnp
from jax.experimental import pallas as pl
from jax.experimental.pallas import tpu as pltpu

NUM_EXPERTS = 64
TOP_K = 8
HIDDEN = 4096
BT = 1024


def _router_block(x_ref, wt_ref, logits_ref, weights_ref, indices_ref):
    x = x_ref[...]
    wt = wt_ref[...]
    logits = jnp.dot(x, wt, preferred_element_type=jnp.float32)
    logits_ref[...] = logits

    lt = logits.T
    m = jnp.max(lt, axis=0, keepdims=True)
    e = jnp.exp(lt - m)

    eidx = jax.lax.broadcasted_iota(jnp.int32, lt.shape, 0)
    fx = (e * jnp.float32(16777216.0)).astype(jnp.int32)
    enc = (fx << 6) | (NUM_EXPERTS - 1 - eidx)

    best = []
    for _ in range(TOP_K):
        b = jnp.max(enc, axis=0, keepdims=True)
        best.append(b)
        enc = jnp.where(enc == b, jnp.int32(-2147483648), enc)

    packed = jnp.concatenate(best, axis=0)
    idx = (NUM_EXPERTS - 1) - (packed & 0x3F)
    vals = (packed >> 6).astype(jnp.float32) * jnp.float32(1.0 / 16777216.0)
    weights_ref[...] = vals / jnp.sum(vals, axis=0, keepdims=True)
    indices_ref[...] = idx


@jax.jit
def kernel(hidden_states, weight):
    x = hidden_states.reshape(-1, HIDDEN)
    n = x.shape[0]
    wt = weight.T
    grid = (n // BT,)
    logits, weights_t, indices_t = pl.pallas_call(
        _router_block,
        grid=grid,
        in_specs=[
            pl.BlockSpec((BT, HIDDEN), lambda i: (i, 0)),
            pl.BlockSpec((HIDDEN, NUM_EXPERTS), lambda i: (0, 0)),
        ],
        out_specs=[
            pl.BlockSpec((BT, NUM_EXPERTS), lambda i: (i, 0)),
            pl.BlockSpec((TOP_K, BT), lambda i: (0, i)),
            pl.BlockSpec((TOP_K, BT), lambda i: (0, i)),
        ],
        out_shape=[
            jax.ShapeDtypeStruct((n, NUM_EXPERTS), jnp.float32),
            jax.ShapeDtypeStruct((TOP_K, n), jnp.float32),
            jax.ShapeDtypeStruct((TOP_K, n), jnp.int32),
        ],
        compiler_params=pltpu.CompilerParams(
            dimension_semantics=("arbitrary",),
        ),
    )(x, wt)
    return logits, weights_t.T, indices_t.T


# speedup vs baseline: 1.0030x; 1.0030x over previous
import jax
import jax.numpy as jnp
from jax.experimental import pallas as pl
from jax.experimental.pallas import tpu as pltpu

NUM_EXPERTS = 64
TOP_K = 8
HIDDEN = 4096
BT = 1024


def _router_block(x_ref, wt_ref, logits_ref, weights_ref, indices_ref):
    x = x_ref[...]
    wt = wt_ref[...]
    logits = jnp.dot(x, wt, preferred_element_type=jnp.float32)
    logits_ref[...] = logits

    lt = logits.T
    m = jnp.max(lt, axis=0, keepdims=True)
    e = jnp.exp(lt - m)

    eidx = jax.lax.broadcasted_iota(jnp.int32, lt.shape, 0)
    fx = (e * jnp.float32(33554368.0)).astype(jnp.int32)
    enc = (fx << 6) | (NUM_EXPERTS - 1 - eidx)

    best = []
    for _ in range(TOP_K):
        b = jnp.max(enc, axis=0, keepdims=True)
        best.append(b)
        enc = jnp.where(enc == b, jnp.int32(-2147483648), enc)

    packed = jnp.concatenate(best, axis=0)
    idx = (NUM_EXPERTS - 1) - (packed & 0x3F)
    vals = (packed >> 6).astype(jnp.float32) * jnp.float32(1.0 / 33554368.0)
    weights_ref[...] = vals / jnp.sum(vals, axis=0, keepdims=True)
    indices_ref[...] = idx


@jax.jit
def kernel(hidden_states, weight):
    x = hidden_states.reshape(-1, HIDDEN)
    n = x.shape[0]
    wt = weight.T
    grid = (n // BT,)
    logits, weights_t, indices_t = pl.pallas_call(
        _router_block,
        grid=grid,
        in_specs=[
            pl.BlockSpec((BT, HIDDEN), lambda i: (i, 0)),
            pl.BlockSpec((HIDDEN, NUM_EXPERTS), lambda i: (0, 0)),
        ],
        out_specs=[
            pl.BlockSpec((BT, NUM_EXPERTS), lambda i: (i, 0)),
            pl.BlockSpec((TOP_K, BT), lambda i: (0, i)),
            pl.BlockSpec((TOP_K, BT), lambda i: (0, i)),
        ],
        out_shape=[
            jax.ShapeDtypeStruct((n, NUM_EXPERTS), jnp.float32),
            jax.ShapeDtypeStruct((TOP_K, n), jnp.float32),
            jax.ShapeDtypeStruct((TOP_K, n), jnp.int32),
        ],
        compiler_params=pltpu.CompilerParams(
            dimension_semantics=("arbitrary",),
        ),
    )(x, wt)
    return logits, weights_t.T, indices_t.T
